# baseline (speedup 1.0000x reference)
import functools

import jax
import jax.numpy as jnp
from jax import lax
from jax.experimental import pallas as pl
from jax.experimental.pallas import tpu as pltpu
from jax.experimental.pallas import tpu_sc as plsc

_LB = 1024
_K = 4
_NC, _NS = 2, 16
_NW = _NC * _NS


def _topk_proj_body(L, NL, x_ref, lut_ref, wt_ref, ids_ref, plut_ref,
                    gidx_ref, s_s, xq_s, vals_s, idx_s):
    b = pl.program_id(0)
    l = pl.program_id(1)
    T = x_ref.shape[2]
    LB = lut_ref.shape[2]

    x_blk = x_ref[0]
    lut_blk = lut_ref[0]
    wt = wt_ref[...]

    plut_ref[0] = lax.dot_general(
        lut_blk, wt, (((0,), (0,)), ((), ())),
        preferred_element_type=jnp.float32)

    NEGF = jnp.float32(jnp.finfo(jnp.float32).min)
    BIG = jnp.int32(jnp.iinfo(jnp.int32).max)

    @pl.when(l == 0)
    def _():
        xn = jnp.sqrt(jnp.sum(x_blk * x_blk, axis=0, keepdims=True))
        xq_s[...] = x_blk / xn
        vals_s[...] = jnp.full(vals_s.shape, NEGF, jnp.float32)
        idx_s[...] = jnp.full(idx_s.shape, BIG, jnp.int32)

    rn = jnp.sqrt(jnp.sum(lut_blk * lut_blk, axis=0, keepdims=True))
    s_s[...] = lax.dot_general(
        lut_blk / rn, xq_s[...], (((0,), (0,)), ((), ())),
        preferred_element_type=jnp.float32)

    ids = ids_ref[...]
    off = l * LB + b * L
    s = s_s[...]
    for k in range(_K):
        m = jnp.max(s, axis=0, keepdims=True)
        sel = jnp.min(jnp.where(s == m, ids, BIG), axis=0, keepdims=True)
        vals_s[pl.ds(_K + k, 1), :] = m
        idx_s[pl.ds(_K + k, 1), :] = sel + off
        if k < _K - 1:
            s = jnp.where(ids == sel, NEGF, s)

    cv = vals_s[...]
    ci = idx_s[...]
    for k in range(_K):
        m = jnp.max(cv, axis=0, keepdims=True)
        sel = jnp.min(jnp.where(cv == m, ci, BIG), axis=0, keepdims=True)
        vals_s[pl.ds(k, 1), :] = m
        idx_s[pl.ds(k, 1), :] = sel
        if k < _K - 1:
            cv = jnp.where(ci == sel, NEGF, cv)

    gidx_ref[0] = idx_s[pl.ds(0, _K), :]


def _topk_and_project(x, lut, wt, interpret=False):
    B, D, T = x.shape
    L = lut.shape[2]
    OP = wt.shape[1]
    NL = L // _LB
    ids_in = jnp.broadcast_to(
        jnp.arange(_LB, dtype=jnp.int32)[:, None], (_LB, T))
    return pl.pallas_call(
        functools.partial(_topk_proj_body, L, NL),
        grid=(B, NL),
        in_specs=[
            pl.BlockSpec((1, D, T), lambda b, l: (b, 0, 0)),
            pl.BlockSpec((1, D, _LB), lambda b, l: (b, 0, l)),
            pl.BlockSpec((D, OP), lambda b, l: (0, 0)),
            pl.BlockSpec((_LB, T), lambda b, l: (0, 0)),
        ],
        out_specs=[
            pl.BlockSpec((1, _LB, OP), lambda b, l: (b, l, 0)),
            pl.BlockSpec((1, _K, T), lambda b, l: (b, 0, 0)),
        ],
        out_shape=[
            jax.ShapeDtypeStruct((B, L, OP), jnp.float32),
            jax.ShapeDtypeStruct((B, _K, T), jnp.int32),
        ],
        scratch_shapes=[
            pltpu.VMEM((_LB, T), jnp.float32),
            pltpu.VMEM((D, T), jnp.float32),
            pltpu.VMEM((2 * _K, T), jnp.float32),
            pltpu.VMEM((2 * _K, T), jnp.int32),
        ],
        interpret=interpret,
    )(x, lut, wt, ids_in)


def _sc_gather_mean(table, gidx, bias):
    n_idx = gidx.shape[0]
    n_out = n_idx // _K
    OP = table.shape[1]
    O = bias.shape[0]
    per_w = n_idx // _NW
    out_per_w = n_out // _NW
    mesh = plsc.VectorSubcoreMesh(core_axis_name="c", subcore_axis_name="s")

    @functools.partial(
        pl.kernel, mesh=mesh,
        out_type=jax.ShapeDtypeStruct((n_out, O), jnp.float32),
        scratch_types=[
            pltpu.VMEM((per_w,), jnp.int32),
            pltpu.VMEM((per_w, OP), jnp.float32),
            pltpu.VMEM((O,), jnp.float32),
            pltpu.VMEM((out_per_w, O), jnp.float32),
            pltpu.SemaphoreType.DMA,
        ],
    )
    def gather_mean(gidx_hbm, table_hbm, bias_hbm, out_hbm,
                    idx_v, rows_v, b_v, acc_v, sem):
        wid = lax.axis_index("s") * _NC + lax.axis_index("c")
        pltpu.sync_copy(bias_hbm, b_v)
        pltpu.sync_copy(gidx_hbm.at[pl.ds(wid * per_w, per_w)], idx_v)
        pltpu.async_copy(table_hbm.at[idx_v], rows_v, sem).wait()

        def body(i, carry):
            for c in range(O // 16):
                sl = pl.ds(c * 16, 16)
                v = (rows_v[_K * i, sl] + rows_v[_K * i + 1, sl]
                     + rows_v[_K * i + 2, sl] + rows_v[_K * i + 3, sl])
                acc_v[i, sl] = v * 0.25 + b_v[sl]
            return carry
        lax.fori_loop(0, out_per_w, body, 0)
        pltpu.sync_copy(acc_v, out_hbm.at[pl.ds(wid * out_per_w, out_per_w)])

    return gather_mean(gidx, table, bias)


def kernel(x, lut, W, b):
    B, D, T = x.shape
    L = lut.shape[2]
    O = W.shape[0]
    OP = 128
    wt = jnp.pad(jnp.transpose(W, (1, 0)), ((0, 0), (0, OP - O)))
    plut, gidx = _topk_and_project(x, lut, wt)
    gidx_t = jnp.transpose(gidx, (0, 2, 1))
    out_bt = _sc_gather_mean(plut.reshape(B * L, OP),
                             gidx_t.reshape(B * T * _K), b)
    return out_bt.reshape(B, T, O).transpose(0, 2, 1)


# speedup vs baseline: 1.2017x; 1.2017x over previous
import functools

import jax
import jax.numpy as jnp
from jax import lax
from jax.experimental import pallas as pl
from jax.experimental.pallas import tpu as pltpu
from jax.experimental.pallas import tpu_sc as plsc

_LB = 2048
_K = 4
_NC, _NS = 2, 16
_NW = _NC * _NS


def _topk_proj_body(L, NL, x_ref, lut_ref, wt_ref, ids_ref, plut_ref,
                    gidx_ref, s_s, vals_s, idx_s):
    b = pl.program_id(0)
    l = pl.program_id(1)
    T = x_ref.shape[2]
    LB = lut_ref.shape[2]

    x_blk = x_ref[0]
    lut_blk = lut_ref[0]
    wt = wt_ref[...]

    plut_ref[0] = lax.dot_general(
        lut_blk, wt, (((0,), (0,)), ((), ())),
        preferred_element_type=jnp.float32)

    rn = jnp.sqrt(jnp.sum(lut_blk * lut_blk, axis=0, keepdims=True))
    xn = jnp.sqrt(jnp.sum(x_blk * x_blk, axis=0, keepdims=True))
    s_s[...] = lax.dot_general(
        lut_blk / rn, x_blk / xn, (((0,), (0,)), ((), ())),
        preferred_element_type=jnp.float32)

    NEGF = jnp.float32(jnp.finfo(jnp.float32).min)
    BIG = jnp.int32(jnp.iinfo(jnp.int32).max)

    @pl.when(l == 0)
    def _():
        vals_s[...] = jnp.full(vals_s.shape, NEGF, jnp.float32)
        idx_s[...] = jnp.full(idx_s.shape, BIG, jnp.int32)

    ids = ids_ref[...]
    off = l * LB + b * L
    s = s_s[...]
    for k in range(_K):
        m = jnp.max(s, axis=0, keepdims=True)
        sel = jnp.min(jnp.where(s == m, ids, BIG), axis=0, keepdims=True)
        vals_s[pl.ds(_K + k, 1), :] = m
        idx_s[pl.ds(_K + k, 1), :] = sel + off
        if k < _K - 1:
            s = jnp.where(ids == sel, NEGF, s)

    cv = vals_s[...]
    ci = idx_s[...]
    for k in range(_K):
        m = jnp.max(cv, axis=0, keepdims=True)
        sel = jnp.min(jnp.where(cv == m, ci, BIG), axis=0, keepdims=True)
        vals_s[pl.ds(k, 1), :] = m
        idx_s[pl.ds(k, 1), :] = sel
        if k < _K - 1:
            cv = jnp.where(ci == sel, NEGF, cv)

    gidx_ref[0] = idx_s[pl.ds(0, _K), :]


def _topk_and_project(x, lut, wt, interpret=False):
    B, D, T = x.shape
    L = lut.shape[2]
    OP = wt.shape[1]
    NL = L // _LB
    ids_in = jnp.broadcast_to(
        jnp.arange(_LB, dtype=jnp.int32)[:, None], (_LB, T))
    return pl.pallas_call(
        functools.partial(_topk_proj_body, L, NL),
        grid=(B, NL),
        in_specs=[
            pl.BlockSpec((1, D, T), lambda b, l: (b, 0, 0)),
            pl.BlockSpec((1, D, _LB), lambda b, l: (b, 0, l)),
            pl.BlockSpec((D, OP), lambda b, l: (0, 0)),
            pl.BlockSpec((_LB, T), lambda b, l: (0, 0)),
        ],
        out_specs=[
            pl.BlockSpec((1, _LB, OP), lambda b, l: (b, l, 0)),
            pl.BlockSpec((1, _K, T), lambda b, l: (b, 0, 0)),
        ],
        out_shape=[
            jax.ShapeDtypeStruct((B, L, OP), jnp.float32),
            jax.ShapeDtypeStruct((B, _K, T), jnp.int32),
        ],
        scratch_shapes=[
            pltpu.VMEM((_LB, T), jnp.float32),
            pltpu.VMEM((2 * _K, T), jnp.float32),
            pltpu.VMEM((2 * _K, T), jnp.int32),
        ],
        interpret=interpret,
    )(x, lut, wt, ids_in)


def _sc_gather_mean(table, gidx, bias):
    n_idx = gidx.shape[0]
    n_out = n_idx // _K
    OP = table.shape[1]
    O = bias.shape[0]
    per_w = n_idx // _NW
    out_per_w = n_out // _NW
    mesh = plsc.VectorSubcoreMesh(core_axis_name="c", subcore_axis_name="s")

    @functools.partial(
        pl.kernel, mesh=mesh,
        out_type=jax.ShapeDtypeStruct((n_out, O), jnp.float32),
        scratch_types=[
            pltpu.VMEM((per_w,), jnp.int32),
            pltpu.VMEM((per_w, OP), jnp.float32),
            pltpu.VMEM((O,), jnp.float32),
            pltpu.VMEM((out_per_w, O), jnp.float32),
            pltpu.SemaphoreType.DMA,
        ],
    )
    def gather_mean(gidx_hbm, table_hbm, bias_hbm, out_hbm,
                    idx_v, rows_v, b_v, acc_v, sem):
        wid = lax.axis_index("s") * _NC + lax.axis_index("c")
        pltpu.sync_copy(bias_hbm, b_v)
        pltpu.sync_copy(gidx_hbm.at[pl.ds(wid * per_w, per_w)], idx_v)
        pltpu.async_copy(table_hbm.at[idx_v], rows_v, sem).wait()

        def body(i, carry):
            for c in range(O // 16):
                sl = pl.ds(c * 16, 16)
                v = (rows_v[_K * i, sl] + rows_v[_K * i + 1, sl]
                     + rows_v[_K * i + 2, sl] + rows_v[_K * i + 3, sl])
                acc_v[i, sl] = v * 0.25 + b_v[sl]
            return carry
        lax.fori_loop(0, out_per_w, body, 0)
        pltpu.sync_copy(acc_v, out_hbm.at[pl.ds(wid * out_per_w, out_per_w)])

    return gather_mean(gidx, table, bias)


def kernel(x, lut, W, b):
    B, D, T = x.shape
    L = lut.shape[2]
    O = W.shape[0]
    OP = 128
    wt = jnp.pad(jnp.transpose(W, (1, 0)), ((0, 0), (0, OP - O)))
    plut, gidx = _topk_and_project(x, lut, wt)
    gidx_t = jnp.transpose(gidx, (0, 2, 1))
    out_bt = _sc_gather_mean(plut.reshape(B * L, OP),
                             gidx_t.reshape(B * T * _K), b)
    return out_bt.reshape(B, T, O).transpose(0, 2, 1)


# speedup vs baseline: 1.2096x; 1.0066x over previous
import functools

import jax
import jax.numpy as jnp
from jax import lax
from jax.experimental import pallas as pl
from jax.experimental.pallas import tpu as pltpu
from jax.experimental.pallas import tpu_sc as plsc

_LB = 4096
_K = 4
_NC, _NS = 2, 16
_NW = _NC * _NS


def _topk_proj_body(L, NL, x_ref, lut_ref, wt_ref, ids_ref, plut_ref,
                    gidx_ref, s_s, vals_s, idx_s):
    b = pl.program_id(0)
    l = pl.program_id(1)
    T = x_ref.shape[2]
    LB = lut_ref.shape[2]

    x_blk = x_ref[0]
    lut_blk = lut_ref[0]
    wt = wt_ref[...]

    plut_ref[0] = lax.dot_general(
        lut_blk, wt, (((0,), (0,)), ((), ())),
        preferred_element_type=jnp.float32)

    rn = jnp.sqrt(jnp.sum(lut_blk * lut_blk, axis=0, keepdims=True))
    xn = jnp.sqrt(jnp.sum(x_blk * x_blk, axis=0, keepdims=True))
    s_s[...] = lax.dot_general(
        lut_blk / rn, x_blk / xn, (((0,), (0,)), ((), ())),
        preferred_element_type=jnp.float32)

    NEGF = jnp.float32(jnp.finfo(jnp.float32).min)
    BIG = jnp.int32(jnp.iinfo(jnp.int32).max)

    @pl.when(l == 0)
    def _():
        vals_s[...] = jnp.full(vals_s.shape, NEGF, jnp.float32)
        idx_s[...] = jnp.full(idx_s.shape, BIG, jnp.int32)

    ids = ids_ref[...]
    off = l * LB + b * L
    s = s_s[...]
    for k in range(_K):
        m = jnp.max(s, axis=0, keepdims=True)
        sel = jnp.min(jnp.where(s == m, ids, BIG), axis=0, keepdims=True)
        vals_s[pl.ds(_K + k, 1), :] = m
        idx_s[pl.ds(_K + k, 1), :] = sel + off
        if k < _K - 1:
            s = jnp.where(ids == sel, NEGF, s)

    cv = vals_s[...]
    ci = idx_s[...]
    for k in range(_K):
        m = jnp.max(cv, axis=0, keepdims=True)
        sel = jnp.min(jnp.where(cv == m, ci, BIG), axis=0, keepdims=True)
        vals_s[pl.ds(k, 1), :] = m
        idx_s[pl.ds(k, 1), :] = sel
        if k < _K - 1:
            cv = jnp.where(ci == sel, NEGF, cv)

    gidx_ref[0] = idx_s[pl.ds(0, _K), :]


def _topk_and_project(x, lut, wt, interpret=False):
    B, D, T = x.shape
    L = lut.shape[2]
    OP = wt.shape[1]
    NL = L // _LB
    ids_in = jnp.broadcast_to(
        jnp.arange(_LB, dtype=jnp.int32)[:, None], (_LB, T))
    return pl.pallas_call(
        functools.partial(_topk_proj_body, L, NL),
        grid=(B, NL),
        in_specs=[
            pl.BlockSpec((1, D, T), lambda b, l: (b, 0, 0)),
            pl.BlockSpec((1, D, _LB), lambda b, l: (b, 0, l)),
            pl.BlockSpec((D, OP), lambda b, l: (0, 0)),
            pl.BlockSpec((_LB, T), lambda b, l: (0, 0)),
        ],
        out_specs=[
            pl.BlockSpec((1, _LB, OP), lambda b, l: (b, l, 0)),
            pl.BlockSpec((1, _K, T), lambda b, l: (b, 0, 0)),
        ],
        out_shape=[
            jax.ShapeDtypeStruct((B, L, OP), jnp.float32),
            jax.ShapeDtypeStruct((B, _K, T), jnp.int32),
        ],
        scratch_shapes=[
            pltpu.VMEM((_LB, T), jnp.float32),
            pltpu.VMEM((2 * _K, T), jnp.float32),
            pltpu.VMEM((2 * _K, T), jnp.int32),
        ],
        interpret=interpret,
    )(x, lut, wt, ids_in)


def _sc_gather_mean(table, gidx, bias):
    n_idx = gidx.shape[0]
    n_out = n_idx // _K
    OP = table.shape[1]
    O = bias.shape[0]
    per_w = n_idx // _NW
    out_per_w = n_out // _NW
    mesh = plsc.VectorSubcoreMesh(core_axis_name="c", subcore_axis_name="s")

    @functools.partial(
        pl.kernel, mesh=mesh,
        out_type=jax.ShapeDtypeStruct((n_out, O), jnp.float32),
        scratch_types=[
            pltpu.VMEM((per_w,), jnp.int32),
            pltpu.VMEM((per_w, OP), jnp.float32),
            pltpu.VMEM((O,), jnp.float32),
            pltpu.VMEM((out_per_w, O), jnp.float32),
            pltpu.SemaphoreType.DMA,
        ],
    )
    def gather_mean(gidx_hbm, table_hbm, bias_hbm, out_hbm,
                    idx_v, rows_v, b_v, acc_v, sem):
        wid = lax.axis_index("s") * _NC + lax.axis_index("c")
        pltpu.sync_copy(bias_hbm, b_v)
        pltpu.sync_copy(gidx_hbm.at[pl.ds(wid * per_w, per_w)], idx_v)
        pltpu.async_copy(table_hbm.at[idx_v], rows_v, sem).wait()

        def body(i, carry):
            for c in range(O // 16):
                sl = pl.ds(c * 16, 16)
                v = (rows_v[_K * i, sl] + rows_v[_K * i + 1, sl]
                     + rows_v[_K * i + 2, sl] + rows_v[_K * i + 3, sl])
                acc_v[i, sl] = v * 0.25 + b_v[sl]
            return carry
        lax.fori_loop(0, out_per_w, body, 0)
        pltpu.sync_copy(acc_v, out_hbm.at[pl.ds(wid * out_per_w, out_per_w)])

    return gather_mean(gidx, table, bias)


def kernel(x, lut, W, b):
    B, D, T = x.shape
    L = lut.shape[2]
    O = W.shape[0]
    OP = 128
    wt = jnp.pad(jnp.transpose(W, (1, 0)), ((0, 0), (0, OP - O)))
    plut, gidx = _topk_and_project(x, lut, wt)
    gidx_t = jnp.transpose(gidx, (0, 2, 1))
    out_bt = _sc_gather_mean(plut.reshape(B * L, OP),
                             gidx_t.reshape(B * T * _K), b)
    return out_bt.reshape(B, T, O).transpose(0, 2, 1)
